# Initial kernel scaffold; baseline (speedup 1.0000x reference)
import jax
import jax.numpy as jnp
from jax.experimental import pallas as pl


def kernel(c, c_edge, c_attrib, c_batch, s, s_edge, s_attrib, s_batch, cro_W, cro_b, solv_W, solv_b, c0_W, c0_b, s0_W, s0_b, c1_W, c1_b, c1_g, c1_bt, c2_W, c2_b, c2_g, c2_bt, s1_W, s1_b, s1_g, s1_bt, s2_W, s2_b, s2_g, s2_bt, dense_W, dense_b, out_W, out_b):
    raise NotImplementedError("write your pallas kernel here")


# speedup vs baseline: 13.2664x; 13.2664x over previous
import functools

import jax
import jax.numpy as jnp
from jax import lax
from jax.experimental import pallas as pl
from jax.experimental.pallas import tpu as pltpu
from jax.experimental.pallas import tpu_sc as plsc

N = 10000
E = 320000
G = 256
DC = 64
DS = 32
NCORE = 2
NSUB = 16
NW = NCORE * NSUB
EW = E // NW
CH = 128
NCH = EW // CH
TAIL = EW - NCH * CH
EW2 = E // NSUB
NCH2 = EW2 // CH
TAIL2 = EW2 - NCH2 * CH
RPT = 624
RLEFT = N - RPT * NSUB

RT = 1000
NB = N // RT

_F32 = jnp.float32


def _mesh():
    return plsc.VectorSubcoreMesh(core_axis_name="c", subcore_axis_name="s")


def _dot(a, b):
    return jnp.dot(a, b, preferred_element_type=_F32)


def _dot_hi(a, b):
    return jnp.dot(a, b, precision=lax.Precision.HIGHEST,
                   preferred_element_type=_F32)


def _striped(sid, fn):
    fn(sid * RPT, RPT)

    @pl.when(sid < 2)
    def _extra():
        fn(RPT * NSUB + sid * 8, 8)


def _deg_body(dst2_hbm, zeros_hbm, ones_hbm, out_hbm,
              acc, idx_v, idxt_v, ones_v, onest_v):
    cid = lax.axis_index("c")
    sid = lax.axis_index("s")
    _striped(sid, lambda r, n: pltpu.sync_copy(
        zeros_hbm.at[pl.ds(r, n)], acc.at[pl.ds(r, n)]))
    pltpu.sync_copy(ones_hbm, ones_v)
    pltpu.sync_copy(ones_hbm.at[pl.ds(0, TAIL2)], onest_v)
    plsc.subcore_barrier()
    base = cid * E + sid * EW2

    @pl.loop(0, NCH2)
    def _chunks(j):
        pltpu.sync_copy(dst2_hbm.at[pl.ds(base + j * CH, CH)], idx_v)
        pltpu.sync_copy(ones_v, acc.at[idx_v], add=True)

    pltpu.sync_copy(dst2_hbm.at[pl.ds(base + NCH2 * CH, TAIL2)], idxt_v)
    pltpu.sync_copy(onest_v, acc.at[idxt_v], add=True)
    plsc.subcore_barrier()
    _striped(sid, lambda r, n: pltpu.sync_copy(
        acc.at[pl.ds(r, n)], out_hbm.at[pl.ds(cid * N + r, n)]))


def _sc_degree(c_dst, s_dst):
    dst2 = jnp.concatenate([c_dst, s_dst])
    zeros8 = jnp.zeros((N, 8), _F32)
    ones8 = jnp.ones((CH, 8), _F32)
    kern = pl.kernel(
        _deg_body,
        out_type=jax.ShapeDtypeStruct((2 * N, 8), _F32),
        mesh=_mesh(),
        scratch_types=[
            pltpu.VMEM_SHARED((N, 8), _F32),
            pltpu.VMEM((CH,), jnp.int32),
            pltpu.VMEM((TAIL2,), jnp.int32),
            pltpu.VMEM((CH, 8), _F32),
            pltpu.VMEM((TAIL2, 8), _F32),
        ],
        compiler_params=pltpu.CompilerParams(use_tc_tiling_on_sc=False))
    return kern(dst2, zeros8, ones8)


def _scat_body(gc_hbm, gs_hbm, csrc, cdst, ssrc, sdst, zc_hbm, zs_hbm,
               outc_hbm, outs_hbm,
               accc, accs, src_v, dst_v, rowsc_v, rowss_v,
               srct_v, dstt_v, rowsct_v, rowsst_v, sem):
    cid = lax.axis_index("c")
    sid = lax.axis_index("s")
    w = cid * NSUB + sid
    _striped(sid, lambda r, n: pltpu.sync_copy(
        zc_hbm.at[pl.ds(r, n)], accc.at[pl.ds(r, n)]))
    _striped(sid, lambda r, n: pltpu.sync_copy(
        zs_hbm.at[pl.ds(r, n)], accs.at[pl.ds(r, n)]))
    plsc.subcore_barrier()
    base = w * EW

    @pl.loop(0, NCH)
    def _cchunks(j):
        off = base + j * CH
        pltpu.sync_copy(csrc.at[pl.ds(off, CH)], src_v)
        pltpu.sync_copy(cdst.at[pl.ds(off, CH)], dst_v)
        pltpu.async_copy(gc_hbm.at[src_v], rowsc_v, sem).wait()
        pltpu.sync_copy(rowsc_v, accc.at[dst_v], add=True)

    offt = base + NCH * CH
    pltpu.sync_copy(csrc.at[pl.ds(offt, TAIL)], srct_v)
    pltpu.sync_copy(cdst.at[pl.ds(offt, TAIL)], dstt_v)
    pltpu.async_copy(gc_hbm.at[srct_v], rowsct_v, sem).wait()
    pltpu.sync_copy(rowsct_v, accc.at[dstt_v], add=True)

    @pl.loop(0, NCH)
    def _schunks(j):
        off = base + j * CH
        pltpu.sync_copy(ssrc.at[pl.ds(off, CH)], src_v)
        pltpu.sync_copy(sdst.at[pl.ds(off, CH)], dst_v)
        pltpu.async_copy(gs_hbm.at[src_v], rowss_v, sem).wait()
        pltpu.sync_copy(rowss_v, accs.at[dst_v], add=True)

    pltpu.sync_copy(ssrc.at[pl.ds(offt, TAIL)], srct_v)
    pltpu.sync_copy(sdst.at[pl.ds(offt, TAIL)], dstt_v)
    pltpu.async_copy(gs_hbm.at[srct_v], rowsst_v, sem).wait()
    pltpu.sync_copy(rowsst_v, accs.at[dstt_v], add=True)

    plsc.subcore_barrier()
    _striped(sid, lambda r, n: pltpu.sync_copy(
        accc.at[pl.ds(r, n)], outc_hbm.at[pl.ds(cid * N + r, n)]))
    _striped(sid, lambda r, n: pltpu.sync_copy(
        accs.at[pl.ds(r, n)], outs_hbm.at[pl.ds(cid * N + r, n)]))


def _sc_scatter(g_c, g_s, c_src, c_dst, s_src, s_dst):
    zc = jnp.zeros((N, DC), _F32)
    zs = jnp.zeros((N, DS), _F32)
    kern = pl.kernel(
        _scat_body,
        out_type=(jax.ShapeDtypeStruct((2 * N, DC), _F32),
                  jax.ShapeDtypeStruct((2 * N, DS), _F32)),
        mesh=_mesh(),
        scratch_types=[
            pltpu.VMEM_SHARED((N, DC), _F32),
            pltpu.VMEM_SHARED((N, DS), _F32),
            pltpu.VMEM((CH,), jnp.int32),
            pltpu.VMEM((CH,), jnp.int32),
            pltpu.VMEM((CH, DC), _F32),
            pltpu.VMEM((CH, DS), _F32),
            pltpu.VMEM((TAIL,), jnp.int32),
            pltpu.VMEM((TAIL,), jnp.int32),
            pltpu.VMEM((TAIL, DC), _F32),
            pltpu.VMEM((TAIL, DS), _F32),
            pltpu.SemaphoreType.DMA,
        ],
        compiler_params=pltpu.CompilerParams(use_tc_tiling_on_sc=False))
    return kern(g_c, g_s, c_src, c_dst, s_src, s_dst, zc, zs)


def _prep_body(degc_ref, c_ref, croW_ref, crob_ref, c0W_ref,
               degs_ref, s_ref, solvW_ref, solvb_ref, s0W_ref,
               dinvc_ref, g0c_ref, dinvs_ref, g0s_ref):
    dc = lax.rsqrt(degc_ref[...] + 1.0)
    xc = _dot(c_ref[...], croW_ref[...]) + crob_ref[...]
    g0c_ref[...] = _dot(xc, c0W_ref[...]) * dc
    dinvc_ref[...] = dc
    ds_ = lax.rsqrt(degs_ref[...] + 1.0)
    xs = _dot(s_ref[...], solvW_ref[...]) + solvb_ref[...]
    g0s_ref[...] = _dot(xs, s0W_ref[...]) * ds_
    dinvs_ref[...] = ds_


def _tc_prep(degc, c, croW, crob, c0W, degs, s, solvW, solvb, s0W):
    row = lambda i: (i, 0)
    full = lambda i: (0, 0)
    return pl.pallas_call(
        _prep_body,
        grid=(NB,),
        in_specs=[
            pl.BlockSpec((RT, 1), row),
            pl.BlockSpec((RT, 128), row),
            pl.BlockSpec((128, DC), full),
            pl.BlockSpec((1, DC), full),
            pl.BlockSpec((DC, DC), full),
            pl.BlockSpec((RT, 1), row),
            pl.BlockSpec((RT, 128), row),
            pl.BlockSpec((128, DS), full),
            pl.BlockSpec((1, DS), full),
            pl.BlockSpec((DS, DS), full),
        ],
        out_specs=[
            pl.BlockSpec((RT, 1), row),
            pl.BlockSpec((RT, DC), row),
            pl.BlockSpec((RT, 1), row),
            pl.BlockSpec((RT, DS), row),
        ],
        out_shape=[
            jax.ShapeDtypeStruct((N, 1), _F32),
            jax.ShapeDtypeStruct((N, DC), _F32),
            jax.ShapeDtypeStruct((N, 1), _F32),
            jax.ShapeDtypeStruct((N, DS), _F32),
        ],
    )(degc, c, croW, crob, c0W, degs, s, solvW, solvb, s0W)


def _ln_relu(x, gam, bet):
    mu = jnp.mean(x, axis=-1, keepdims=True)
    var = jnp.mean((x - mu) ** 2, axis=-1, keepdims=True)
    t = (x - mu) * lax.rsqrt(var + 1e-5) * gam + bet
    return jnp.maximum(t, 0.0)


def _make_mid_body(residual):
    def body(*refs):
        (Sac, Sbc, gc, *rest) = refs
        if residual:
            xc_ref = rest[0]
            rest = rest[1:]
        (dinvc, bc, lngc, lnbc, Wc,
         Sas, Sbs, gs, *rest2) = rest
        if residual:
            xs_ref = rest2[0]
            rest2 = rest2[1:]
        (dinvs, bs, lngs, lnbs, Ws,
         xnc_ref, gnc_ref, xns_ref, gns_ref) = rest2

        xc = dinvc[...] * (Sac[...] + Sbc[...] + gc[...]) + bc[...]
        if residual:
            xc = xc + xc_ref[...]
        tc = _ln_relu(xc, lngc[...], lnbc[...])
        xnc_ref[...] = xc
        gnc_ref[...] = _dot(tc, Wc[...]) * dinvc[...]

        xs = dinvs[...] * (Sas[...] + Sbs[...] + gs[...]) + bs[...]
        if residual:
            xs = xs + xs_ref[...]
        ts = _ln_relu(xs, lngs[...], lnbs[...])
        xns_ref[...] = xs
        gns_ref[...] = _dot(ts, Ws[...]) * dinvs[...]
    return body


def _tc_mid(residual, Sac, Sbc, gc, xc, dinvc, bc, lngc, lnbc, Wc,
            Sas, Sbs, gs, xs, dinvs, bs, lngs, lnbs, Ws):
    row = lambda i: (i, 0)
    full = lambda i: (0, 0)

    def net_specs(d):
        sp = [pl.BlockSpec((RT, d), row)] * 3
        if residual:
            sp.append(pl.BlockSpec((RT, d), row))
        sp += [
            pl.BlockSpec((RT, 1), row),
            pl.BlockSpec((1, d), full),
            pl.BlockSpec((1, d), full),
            pl.BlockSpec((1, d), full),
            pl.BlockSpec((d, d), full),
        ]
        return sp

    in_specs = net_specs(DC) + net_specs(DS)
    argsc = [Sac, Sbc, gc] + ([xc] if residual else []) + [dinvc, bc, lngc, lnbc, Wc]
    argss = [Sas, Sbs, gs] + ([xs] if residual else []) + [dinvs, bs, lngs, lnbs, Ws]
    return pl.pallas_call(
        _make_mid_body(residual),
        grid=(NB,),
        in_specs=in_specs,
        out_specs=[
            pl.BlockSpec((RT, DC), row),
            pl.BlockSpec((RT, DC), row),
            pl.BlockSpec((RT, DS), row),
            pl.BlockSpec((RT, DS), row),
        ],
        out_shape=[
            jax.ShapeDtypeStruct((N, DC), _F32),
            jax.ShapeDtypeStruct((N, DC), _F32),
            jax.ShapeDtypeStruct((N, DS), _F32),
            jax.ShapeDtypeStruct((N, DS), _F32),
        ],
    )(*(argsc + argss))


WIN = 32
RC = 200
GPAD = 288


def _pool_net(x3, ids_row, ids_col, g_lo, g_hi, sum_ref, cnt_ref, max_ref, d):
    oh = (lax.broadcasted_iota(jnp.int32, (G, RT), 0) == ids_row).astype(_F32)
    sum_ref[...] += _dot_hi(oh, x3)
    cnt_ref[...] += jnp.sum(oh, axis=1, keepdims=True)

    base = (g_lo // 8) * 8
    npass = (g_hi - base) // WIN + 1

    def pbody(k, carry):
        wstart = base + k * WIN
        parts = []
        for w in range(WIN):
            mask_w = ids_col == (wstart + w)
            xw = jnp.where(mask_w, x3, -jnp.inf)
            parts.append(jnp.max(xw, axis=0, keepdims=True))
        part = jnp.concatenate(parts, axis=0)
        cur = max_ref[pl.ds(wstart, WIN), :]
        max_ref[pl.ds(wstart, WIN), :] = jnp.maximum(cur, part)
        return carry

    lax.fori_loop(0, npass, pbody, 0)


def _pool_body(Sac, Sbc, gc, xc, dinvc, bc, idsc_ref, idscc_ref,
               Sas, Sbs, gs, xs, dinvs, bs, idss_ref, idssc_ref,
               dW, db, oW, ob,
               out_ref, emb_ref,
               sumc, cntc, maxc, sums, cnts, maxs):
    step = pl.program_id(0)

    @pl.when(step == 0)
    def _init():
        sumc[...] = jnp.zeros((G, DC), _F32)
        cntc[...] = jnp.zeros((G, 1), _F32)
        maxc[...] = jnp.full((GPAD, DC), -jnp.inf, _F32)
        sums[...] = jnp.zeros((G, DS), _F32)
        cnts[...] = jnp.zeros((G, 1), _F32)
        maxs[...] = jnp.full((GPAD, DS), -jnp.inf, _F32)

    x3c = xc[...] + dinvc[...] * (Sac[...] + Sbc[...] + gc[...]) + bc[...]
    x3s = xs[...] + dinvs[...] * (Sas[...] + Sbs[...] + gs[...]) + bs[...]
    _pool_net(x3c, idsc_ref[0], idscc_ref[...], idscc_ref[0, 0],
              idscc_ref[RT - 1, 0], sumc, cntc, maxc, DC)
    _pool_net(x3s, idss_ref[0], idssc_ref[...], idssc_ref[0, 0],
              idssc_ref[RT - 1, 0], sums, cnts, maxs, DS)

    @pl.when(step == NB - 1)
    def _fin():
        gmp_c = jnp.where(cntc[...] > 0, maxc[0:G, :], 0.0)
        gap_c = sumc[...] / jnp.maximum(cntc[...], 1.0)
        gmp_s = jnp.where(cnts[...] > 0, maxs[0:G, :], 0.0)
        gap_s = sums[...] / jnp.maximum(cnts[...], 1.0)
        embed = jnp.concatenate([gmp_c, gap_c, gmp_s, gap_s], axis=1)
        dense = jnp.maximum(_dot(embed, dW[...]) + db[...], 0.0)
        out_ref[...] = _dot(dense, oW[...]) + ob[...]
        emb_ref[...] = embed


def _tc_pool(Sac, Sbc, gc, xc, dinvc, bc, idsc, idscc,
             Sas, Sbs, gs, xs, dinvs, bs, idss, idssc,
             dW, db, oW, ob):
    row = lambda i: (i, 0)
    full = lambda i: (0, 0)
    ids_spec = pl.BlockSpec((1, 1, RT), lambda i: (i, 0, 0))

    def net_specs(d):
        return [pl.BlockSpec((RT, d), row)] * 4 + [
            pl.BlockSpec((RT, 1), row),
            pl.BlockSpec((1, d), full),
            ids_spec,
            pl.BlockSpec((RT, 1), row),
        ]

    return pl.pallas_call(
        _pool_body,
        grid=(NB,),
        in_specs=(net_specs(DC) + net_specs(DS)
                  + [pl.BlockSpec((2 * (DC + DS), 128), full),
                     pl.BlockSpec((1, 128), full),
                     pl.BlockSpec((128, 1), full),
                     pl.BlockSpec((1, 1), full)]),
        out_specs=[
            pl.BlockSpec((G, 1), full),
            pl.BlockSpec((G, 2 * (DC + DS)), full),
        ],
        out_shape=[
            jax.ShapeDtypeStruct((G, 1), _F32),
            jax.ShapeDtypeStruct((G, 2 * (DC + DS)), _F32),
        ],
        scratch_shapes=[
            pltpu.VMEM((G, DC), _F32),
            pltpu.VMEM((G, 1), _F32),
            pltpu.VMEM((GPAD, DC), _F32),
            pltpu.VMEM((G, DS), _F32),
            pltpu.VMEM((G, 1), _F32),
            pltpu.VMEM((GPAD, DS), _F32),
        ],
    )(Sac, Sbc, gc, xc, dinvc, bc, idsc, idscc,
      Sas, Sbs, gs, xs, dinvs, bs, idss, idssc,
      dW, db, oW, ob)


def kernel(c, c_edge, c_attrib, c_batch, s, s_edge, s_attrib, s_batch,
           cro_W, cro_b, solv_W, solv_b, c0_W, c0_b, s0_W, s0_b,
           c1_W, c1_b, c1_g, c1_bt, c2_W, c2_b, c2_g, c2_bt,
           s1_W, s1_b, s1_g, s1_bt, s2_W, s2_b, s2_g, s2_bt,
           dense_W, dense_b, out_W, out_b):
    i32 = jnp.int32
    c_src = c_edge[0].astype(i32)
    c_dst = c_edge[1].astype(i32)
    s_src = s_edge[0].astype(i32)
    s_dst = s_edge[1].astype(i32)

    deg2 = _sc_degree(c_dst, s_dst)
    degc = deg2[0:N, 0:1]
    degs = deg2[N:2 * N, 0:1]

    r1 = lambda v: v.reshape(1, -1)
    dinvc, g0c, dinvs, g0s = _tc_prep(
        degc, c, cro_W, r1(cro_b), c0_W,
        degs, s, solv_W, r1(solv_b), s0_W)

    S0c, S0s = _sc_scatter(g0c, g0s, c_src, c_dst, s_src, s_dst)
    x1c, g1c, x1s, g1s = _tc_mid(
        False, S0c[:N], S0c[N:], g0c, None, dinvc, r1(c0_b), r1(c1_g),
        r1(c1_bt), c1_W,
        S0s[:N], S0s[N:], g0s, None, dinvs, r1(s0_b), r1(s1_g),
        r1(s1_bt), s1_W)

    S1c, S1s = _sc_scatter(g1c, g1s, c_src, c_dst, s_src, s_dst)
    x2c, g2c, x2s, g2s = _tc_mid(
        True, S1c[:N], S1c[N:], g1c, x1c, dinvc, r1(c1_b), r1(c2_g),
        r1(c2_bt), c2_W,
        S1s[:N], S1s[N:], g1s, x1s, dinvs, r1(s1_b), r1(s2_g),
        r1(s2_bt), s2_W)

    S2c, S2s = _sc_scatter(g2c, g2s, c_src, c_dst, s_src, s_dst)
    idsc = c_batch.astype(i32).reshape(NB, 1, RT)
    idss = s_batch.astype(i32).reshape(NB, 1, RT)
    idscc = c_batch.astype(i32).reshape(N, 1)
    idssc = s_batch.astype(i32).reshape(N, 1)
    out, embed = _tc_pool(
        S2c[:N], S2c[N:], g2c, x2c, dinvc, r1(c2_b), idsc, idscc,
        S2s[:N], S2s[N:], g2s, x2s, dinvs, r1(s2_b), idss, idssc,
        dense_W, r1(dense_b), out_W, r1(out_b))
    return (out, embed)
